# baseline (speedup 1.0000x reference)
import functools
import math

import jax
import jax.numpy as jnp
from jax import lax
from jax.experimental import pallas as pl
from jax.experimental.pallas import tpu as pltpu
from jax.experimental.pallas import tpu_sc as plsc

VOCAB = 1000000
D = 64
B = 4096 * 200
NC, NS = 2, 16
NW = NC * NS
B_PER_W = B // NW
CH = 128
NCH = B_PER_W // CH
SCALE = math.sqrt(D)


def _emb_call(x_flat, table):
    mesh = plsc.VectorSubcoreMesh(core_axis_name="c", subcore_axis_name="s")

    @functools.partial(
        pl.kernel,
        mesh=mesh,
        out_type=jax.ShapeDtypeStruct((B, D), jnp.float32),
        compiler_params=pltpu.CompilerParams(use_tc_tiling_on_sc=False),
        scratch_types=[
            pltpu.VMEM((B_PER_W,), jnp.int32),
            pltpu.VMEM((CH, D), jnp.float32),
            pltpu.SemaphoreType.DMA,
        ],
    )
    def emb_kernel(idx_hbm, table_hbm, out_hbm, idx_v, rows_v, sem):
        wid = lax.axis_index("s") * NC + lax.axis_index("c")
        base = wid * B_PER_W
        pltpu.sync_copy(idx_hbm.at[pl.ds(base, B_PER_W)], idx_v)

        def chunk_body(g, carry):
            idx_sl = idx_v.at[pl.ds(g * CH, CH)]
            pltpu.async_copy(table_hbm.at[idx_sl], rows_v, sem).wait()

            def scale_body(r, c2):
                for c in range(D // 16):
                    v = rows_v[r, pl.ds(c * 16, 16)]
                    rows_v[r, pl.ds(c * 16, 16)] = v * SCALE
                return c2

            lax.fori_loop(0, CH, scale_body, 0)
            pltpu.sync_copy(rows_v, out_hbm.at[pl.ds(base + g * CH, CH)])
            return carry

        lax.fori_loop(0, NCH, chunk_body, 0)

    return emb_kernel(x_flat, table)


@jax.jit
def kernel(x, table):
    x_flat = x.reshape(-1).astype(jnp.int32)
    out = _emb_call(x_flat, table)
    return out.reshape(x.shape[0], x.shape[1], D)


# speedup vs baseline: 1.2112x; 1.2112x over previous
import functools
import math

import jax
import jax.numpy as jnp
from jax import lax
from jax.experimental import pallas as pl
from jax.experimental.pallas import tpu as pltpu
from jax.experimental.pallas import tpu_sc as plsc

VOCAB = 1000000
D = 64
B = 4096 * 200
NC, NS = 2, 16
NW = NC * NS
B_PER_W = B // NW
CH = 128
NCH = B_PER_W // CH
NBUF = 4
NOUT = NCH // NBUF
SCALE = math.sqrt(D)
RU = 4


def _emb_call(x_flat, table):
    mesh = plsc.VectorSubcoreMesh(core_axis_name="c", subcore_axis_name="s")

    @functools.partial(
        pl.kernel,
        mesh=mesh,
        out_type=jax.ShapeDtypeStruct((B, D), jnp.float32),
        compiler_params=pltpu.CompilerParams(use_tc_tiling_on_sc=False),
        scratch_types=[
            pltpu.VMEM((B_PER_W,), jnp.int32),
            pltpu.VMEM((NBUF, CH, D), jnp.float32),
            pltpu.VMEM((NBUF, CH, D), jnp.float32),
        ]
        + [pltpu.SemaphoreType.DMA] * NBUF
        + [pltpu.SemaphoreType.DMA] * NBUF,
    )
    def emb_kernel(idx_hbm, table_hbm, out_hbm, idx_v, ibuf, obuf, *sems):
        gsem = sems[:NBUF]
        osem = sems[NBUF:]
        wid = lax.axis_index("s") * NC + lax.axis_index("c")
        base = wid * B_PER_W
        pltpu.sync_copy(idx_hbm.at[pl.ds(base, B_PER_W)], idx_v)

        def gather(g, b):
            return pltpu.make_async_copy(
                table_hbm.at[idx_v.at[pl.ds(g * CH, CH)]], ibuf.at[b], gsem[b]
            )

        def out_copy(g, b):
            return pltpu.make_async_copy(
                obuf.at[b], out_hbm.at[pl.ds(base + g * CH, CH)], osem[b]
            )

        def scale(b):
            def sbody(r0, c2):
                for dr in range(RU):
                    for c in range(D // 16):
                        sl = pl.ds(c * 16, 16)
                        obuf[b, r0 * RU + dr, sl] = ibuf[b, r0 * RU + dr, sl] * SCALE
                return c2

            lax.fori_loop(0, CH // RU, sbody, 0)

        for b in range(NBUF):
            gather(b, b).start()
        for b in range(NBUF):
            gather(b, b).wait()
            scale(b)
            out_copy(b, b).start()
            gather(b + NBUF, b).start()

        def outer(o, c2):
            for b in range(NBUF):
                g = o * NBUF + b
                gather(g, b).wait()
                out_copy(g - NBUF, b).wait()
                scale(b)
                out_copy(g, b).start()
                gather(g + NBUF, b).start()
            return c2

        lax.fori_loop(1, NOUT - 1, outer, 0)

        for b in range(NBUF):
            g = (NOUT - 1) * NBUF + b
            gather(g, b).wait()
            out_copy(g - NBUF, b).wait()
            scale(b)
            out_copy(g, b).start()
        for b in range(NBUF):
            out_copy((NOUT - 1) * NBUF + b, b).wait()

    return emb_kernel(x_flat, table)


@jax.jit
def kernel(x, table):
    x_flat = x.reshape(-1).astype(jnp.int32)
    out = _emb_call(x_flat, table)
    return out.reshape(x.shape[0], x.shape[1], D)
